# baseline (speedup 1.0000x reference)
import jax
import jax.numpy as jnp
from jax import lax
from jax.experimental import pallas as pl
from jax.experimental.pallas import tpu as pltpu
from jax.experimental.pallas import tpu_sc as plsc

NC = 2
NS = 16
L = 16
NW = NC * NS

B = 16384
D = 64
BPW = B // NW
EPS = 1e-6

COLS = 7812
TAIL0 = COLS * 128
CPW = 245
CHC = 4
CHROWS = CHC * 128
ECAP = 2048
RCAP = 1024
MCAP = 768
TCAP = 64
RING = 32
SEG = 1024


def _p1_body(trip_hbm, entT_hbm, relT_hbm, etail_hbm, rtail_hbm,
             stg_e_hbm, stg_r_hbm,
             seg_v, chunkbuf, etail_v, rtail_v,
             er_l, ed_l, rr_l, rd_l, ter_l, ted_l, trr_l, trd_l,
             mr_l, md_l, ring,
             sem_f, sem_s):
    wid = lax.axis_index("s") * NC + lax.axis_index("c")
    lo = wid * CPW
    hi = lax.min(lo + CPW, COLS)
    lane = lax.iota(jnp.int32, L)
    dlo = wid * BPW

    pltpu.sync_copy(etail_hbm, etail_v)
    pltpu.sync_copy(rtail_hbm, rtail_v)

    def seg_scan(s, carry):
        pltpu.sync_copy(trip_hbm.at[pl.ds(s * SEG * 3, SEG * 3)], seg_v)

        def scan16(i, c):
            en, rn, ten, trn = c
            t = lane + (s * SEG + i * L)
            tl = lane + i * L
            mine_d = (t >= dlo) & (t < dlo + BPW)
            for col_c, dest_off, kind in ((0, 0, "e"), (1, B, "e"),
                                          (2, 0, "r")):
                r = plsc.load_gather(seg_v, [tl * 3 + col_c])
                colid = lax.shift_right_logical(r, 7)
                m = (colid >= lo) & (colid < hi)
                mt = (r >= TAIL0) & mine_d
                d = t + dest_off
                if kind == "e":
                    plsc.store_compressed(er_l.at[pl.ds(en, L)], r, mask=m)
                    plsc.store_compressed(ed_l.at[pl.ds(en, L)], d, mask=m)
                    en = en + plsc.all_reduce_population_count(m)[0]
                    plsc.store_compressed(ter_l.at[pl.ds(ten, L)], r, mask=mt)
                    plsc.store_compressed(ted_l.at[pl.ds(ten, L)], d, mask=mt)
                    ten = ten + plsc.all_reduce_population_count(mt)[0]
                else:
                    plsc.store_compressed(rr_l.at[pl.ds(rn, L)], r, mask=m)
                    plsc.store_compressed(rd_l.at[pl.ds(rn, L)], d, mask=m)
                    rn = rn + plsc.all_reduce_population_count(m)[0]
                    plsc.store_compressed(trr_l.at[pl.ds(trn, L)], r, mask=mt)
                    plsc.store_compressed(trd_l.at[pl.ds(trn, L)], d, mask=mt)
                    trn = trn + plsc.all_reduce_population_count(mt)[0]
            return en, rn, ten, trn

        return lax.fori_loop(0, SEG // L, scan16, carry)

    en, rn, ten, trn = lax.fori_loop(
        0, B // SEG, seg_scan,
        (jnp.int32(0), jnp.int32(0), jnp.int32(0), jnp.int32(0)))

    def run_stream(tab_hbm, stg_hbm, r_l, d_l, nreq):
        nch = lax.div(hi - lo + (CHC - 1), CHC)

        def chunk(cc, _):
            c0 = lo + cc * CHC
            c0c = lax.min(c0, hi - CHC)
            coff = pl.multiple_of(c0c * 128, 128)
            cp = pltpu.make_async_copy(
                tab_hbm.at[:, pl.ds(coff, CHROWS)], chunkbuf, sem_f)
            cp.start()
            cp.wait()
            lo_r = c0 * 128
            hi_r = lax.min(c0 + CHC, hi) * 128

            def rescan(v, mn):
                rv = r_l[pl.ds(v * L, L)]
                dv = d_l[pl.ds(v * L, L)]
                valid = (lane + v * L) < nreq
                m = valid & (rv >= lo_r) & (rv < hi_r)
                plsc.store_compressed(mr_l.at[pl.ds(mn, L)], rv, mask=m)
                plsc.store_compressed(md_l.at[pl.ds(mn, L)], dv, mask=m)
                return mn + plsc.all_reduce_population_count(m)[0]

            cn = lax.fori_loop(0, lax.div(nreq + (L - 1), L), rescan,
                               jnp.int32(0))

            def emit(p, _):
                rv = mr_l[pl.ds(p, L)][0]
                dv = md_l[pl.ds(p, L)][0]

                @pl.when(p >= RING)
                def _():
                    pltpu.make_async_copy(
                        ring.at[pl.ds(0, D)], stg_hbm.at[pl.ds(0, D)],
                        sem_s).wait()

                col = rv - c0c * 128
                slot = lax.rem(p, RING) * D
                for k in range(D // L):
                    vals = plsc.load_gather(
                        chunkbuf, [lane + k * L, jnp.full((L,), 0, jnp.int32)
                                   + col])
                    ring[pl.ds(slot + k * L, L)] = vals
                pltpu.make_async_copy(
                    ring.at[pl.ds(slot, D)],
                    stg_hbm.at[pl.ds(dv * D, D)], sem_s).start()
                return 0

            lax.fori_loop(0, cn, emit, 0)

            def drain(q, _):
                pltpu.make_async_copy(
                    ring.at[pl.ds(0, D)], stg_hbm.at[pl.ds(0, D)],
                    sem_s).wait()
                return 0

            lax.fori_loop(0, lax.min(cn, RING), drain, 0)
            return 0

        lax.fori_loop(0, nch, chunk, 0)

    run_stream(entT_hbm, stg_e_hbm, er_l, ed_l, en)
    run_stream(relT_hbm, stg_r_hbm, rr_l, rd_l, rn)

    def run_tail(tail_v, stg_hbm, r_l, d_l, ntail):
        def emit(p, _):
            rv = r_l[pl.ds(p, L)][0]
            dv = d_l[pl.ds(p, L)][0]
            base = (rv - TAIL0) * D
            for k in range(D // L):
                vals = plsc.load_gather(tail_v, [lane + (base + k * L)])
                ring[pl.ds(k * L, L)] = vals
            cp = pltpu.make_async_copy(
                ring.at[pl.ds(0, D)], stg_hbm.at[pl.ds(dv * D, D)], sem_s)
            cp.start()
            cp.wait()
            return 0

        lax.fori_loop(0, ntail, emit, 0)

    run_tail(etail_v, stg_e_hbm, ter_l, ted_l, ten)
    run_tail(rtail_v, stg_r_hbm, trr_l, trd_l, trn)


def _p2_body(stg_e_hbm, stg_r_hbm, out_hbm,
             rows_s, rows_o, rows_r, out_v, sem):
    wid = lax.axis_index("s") * NC + lax.axis_index("c")
    base = wid * BPW
    lane = lax.iota(jnp.int32, L)

    cs = pltpu.make_async_copy(
        stg_e_hbm.at[pl.ds(base * D, BPW * D)], rows_s, sem)
    co = pltpu.make_async_copy(
        stg_e_hbm.at[pl.ds((B + base) * D, BPW * D)], rows_o, sem)
    cr = pltpu.make_async_copy(
        stg_r_hbm.at[pl.ds(base * D, BPW * D)], rows_r, sem)
    cs.start()
    co.start()
    cr.start()
    cs.wait()
    co.wait()
    cr.wait()

    def group(g, _):
        flat0 = (lane + g * L) * D
        acc = jnp.zeros((L,), jnp.float32)
        for d in range(D):
            s = plsc.load_gather(rows_s, [flat0 + d])
            o = plsc.load_gather(rows_o, [flat0 + d])
            r = plsc.load_gather(rows_r, [flat0 + d])
            t = (s + r) - o + EPS
            acc = acc + t * t
        xi = plsc.bitcast(acc, jnp.int32)
        y = plsc.bitcast(0x5F3759DF - lax.shift_right_logical(xi, 1),
                         jnp.float32)
        hx = 0.5 * acc
        for _ in range(3):
            y = y * (1.5 - (hx * y) * y)
        out_v[pl.ds(g * L, L)] = acc * y
        return 0

    lax.fori_loop(0, BPW // L, group, 0)
    pltpu.sync_copy(out_v, out_hbm.at[pl.ds(base, BPW)])


def kernel(triples, entity_table, relation_table):
    mesh = plsc.VectorSubcoreMesh(core_axis_name="c", subcore_axis_name="s")
    cp = pltpu.CompilerParams(
        needs_layout_passes=False, use_tc_tiling_on_sc=True)
    triples_flat = triples.reshape(-1)
    entT = entity_table.T
    relT = relation_table.T
    etail = entity_table[TAIL0:].reshape(-1)
    rtail = relation_table[TAIL0:].reshape(-1)

    stg_e, stg_r = pl.kernel(
        _p1_body,
        out_type=[
            jax.ShapeDtypeStruct((2 * B * D,), jnp.float32),
            jax.ShapeDtypeStruct((B * D,), jnp.float32),
        ],
        mesh=mesh,
        compiler_params=cp,
        scratch_types=[
            pltpu.VMEM((SEG * 3,), jnp.int32),
            pltpu.VMEM((D, CHROWS), jnp.float32),
            pltpu.VMEM((64 * D,), jnp.float32),
            pltpu.VMEM((64 * D,), jnp.float32),
            pltpu.VMEM((ECAP,), jnp.int32),
            pltpu.VMEM((ECAP,), jnp.int32),
            pltpu.VMEM((RCAP,), jnp.int32),
            pltpu.VMEM((RCAP,), jnp.int32),
            pltpu.VMEM((TCAP,), jnp.int32),
            pltpu.VMEM((TCAP,), jnp.int32),
            pltpu.VMEM((TCAP,), jnp.int32),
            pltpu.VMEM((TCAP,), jnp.int32),
            pltpu.VMEM((MCAP,), jnp.int32),
            pltpu.VMEM((MCAP,), jnp.int32),
            pltpu.VMEM((RING * D,), jnp.float32),
            pltpu.SemaphoreType.DMA,
            pltpu.SemaphoreType.DMA,
        ],
    )(triples_flat, entT, relT, etail, rtail)

    scores = pl.kernel(
        _p2_body,
        out_type=jax.ShapeDtypeStruct((B,), jnp.float32),
        mesh=mesh,
        compiler_params=cp,
        scratch_types=[
            pltpu.VMEM((BPW * D,), jnp.float32),
            pltpu.VMEM((BPW * D,), jnp.float32),
            pltpu.VMEM((BPW * D,), jnp.float32),
            pltpu.VMEM((BPW,), jnp.float32),
            pltpu.SemaphoreType.DMA,
        ],
    )(stg_e, stg_r)
    return scores


# speedup vs baseline: 1.4205x; 1.4205x over previous
import jax
import jax.numpy as jnp
from jax import lax
from jax.experimental import pallas as pl
from jax.experimental.pallas import tpu as pltpu
from jax.experimental.pallas import tpu_sc as plsc

NC = 2
NS = 16
L = 16
NW = NC * NS

B = 16384
D = 64
BPW = B // NW
EPS = 1e-6

COLS = 7812
TAIL0 = COLS * 128
CPW = 245
CHC = 4
CHROWS = CHC * 128
ECAP = 2048
RCAP = 1024
MCAP = 768
TCAP = 64
RING = 32
SEG = 1024


def _p1_body(trip_hbm, entT_hbm, relT_hbm, etail_hbm, rtail_hbm,
             stg_e_hbm, stg_r_hbm,
             seg_v, chunkbuf0, chunkbuf1, etail_v, rtail_v,
             er_l, ed_l, rr_l, rd_l, ter_l, ted_l, trr_l, trd_l,
             mr_l, md_l, ring,
             sem_f0, sem_f1, sem_s):
    wid = lax.axis_index("s") * NC + lax.axis_index("c")
    lo = wid * CPW
    hi = lax.min(lo + CPW, COLS)
    lane = lax.iota(jnp.int32, L)
    dlo = wid * BPW

    pltpu.sync_copy(etail_hbm, etail_v)
    pltpu.sync_copy(rtail_hbm, rtail_v)

    def seg_scan(s, carry):
        pltpu.sync_copy(trip_hbm.at[pl.ds(s * SEG * 3, SEG * 3)], seg_v)

        def scan16(i, c):
            en, rn = c
            tl = lane + i * L
            t = tl + s * SEG
            for col_c, dest_off, kind in ((0, 0, "e"), (1, B, "e"),
                                          (2, 0, "r")):
                r = plsc.load_gather(seg_v, [tl * 3 + col_c])
                colid = lax.shift_right_logical(r, 7)
                m = (colid >= lo) & (colid < hi)
                d = t + dest_off
                if kind == "e":
                    plsc.store_compressed(er_l.at[pl.ds(en, L)], r, mask=m)
                    plsc.store_compressed(ed_l.at[pl.ds(en, L)], d, mask=m)
                    en = en + plsc.all_reduce_population_count(m)[0]
                else:
                    plsc.store_compressed(rr_l.at[pl.ds(rn, L)], r, mask=m)
                    plsc.store_compressed(rd_l.at[pl.ds(rn, L)], d, mask=m)
                    rn = rn + plsc.all_reduce_population_count(m)[0]
            return en, rn

        return lax.fori_loop(0, SEG // L, scan16, carry)

    en, rn = lax.fori_loop(
        0, B // SEG, seg_scan, (jnp.int32(0), jnp.int32(0)))

    pltpu.sync_copy(trip_hbm.at[pl.ds(dlo * 3, BPW * 3)],
                    seg_v.at[pl.ds(0, BPW * 3)])

    def tail_scan(i, c):
        ten, trn = c
        tl = lane + i * L
        for col_c, dest_off, kind in ((0, 0, "e"), (1, B, "e"), (2, 0, "r")):
            r = plsc.load_gather(seg_v, [tl * 3 + col_c])
            mt = r >= TAIL0
            d = tl + dlo + dest_off
            if kind == "e":
                plsc.store_compressed(ter_l.at[pl.ds(ten, L)], r, mask=mt)
                plsc.store_compressed(ted_l.at[pl.ds(ten, L)], d, mask=mt)
                ten = ten + plsc.all_reduce_population_count(mt)[0]
            else:
                plsc.store_compressed(trr_l.at[pl.ds(trn, L)], r, mask=mt)
                plsc.store_compressed(trd_l.at[pl.ds(trn, L)], d, mask=mt)
                trn = trn + plsc.all_reduce_population_count(mt)[0]
        return ten, trn

    ten, trn = lax.fori_loop(0, BPW // L, tail_scan,
                             (jnp.int32(0), jnp.int32(0)))

    def run_stream(tab_hbm, stg_hbm, r_l, d_l, nreq):
        nch = lax.div(hi - lo + (CHC - 1), CHC)

        def fetch(cc, buf, sem):
            c0c = lax.min(lo + cc * CHC, hi - CHC)
            coff = pl.multiple_of(c0c * 128, 128)
            pltpu.make_async_copy(
                tab_hbm.at[:, pl.ds(coff, CHROWS)], buf, sem).start()

        def fwait(buf, sem):
            pltpu.make_async_copy(
                tab_hbm.at[:, pl.ds(0, CHROWS)], buf, sem).wait()

        def process(cc, buf):
            c0 = lo + cc * CHC
            c0c = lax.min(c0, hi - CHC)
            lo_r = c0 * 128
            hi_r = lax.min(c0 + CHC, hi) * 128

            def rescan(v, mn):
                rv = r_l[pl.ds(v * L, L)]
                dv = d_l[pl.ds(v * L, L)]
                valid = (lane + v * L) < nreq
                m = valid & (rv >= lo_r) & (rv < hi_r)
                plsc.store_compressed(mr_l.at[pl.ds(mn, L)], rv, mask=m)
                plsc.store_compressed(md_l.at[pl.ds(mn, L)], dv, mask=m)
                return mn + plsc.all_reduce_population_count(m)[0]

            cn = lax.fori_loop(0, lax.div(nreq + (L - 1), L), rescan,
                               jnp.int32(0))

            def emit(p, _):
                rv = mr_l[pl.ds(p, L)][0]
                dv = md_l[pl.ds(p, L)][0]

                @pl.when(p >= RING)
                def _():
                    pltpu.make_async_copy(
                        ring.at[pl.ds(0, D)], stg_hbm.at[pl.ds(0, D)],
                        sem_s).wait()

                col = rv - c0c * 128
                slot = lax.rem(p, RING) * D
                for k in range(D // L):
                    vals = plsc.load_gather(
                        buf, [lane + k * L, jnp.full((L,), 0, jnp.int32)
                              + col])
                    ring[pl.ds(slot + k * L, L)] = vals
                pltpu.make_async_copy(
                    ring.at[pl.ds(slot, D)],
                    stg_hbm.at[pl.ds(dv * D, D)], sem_s).start()
                return 0

            lax.fori_loop(0, cn, emit, 0)

            def drain(q, _):
                pltpu.make_async_copy(
                    ring.at[pl.ds(0, D)], stg_hbm.at[pl.ds(0, D)],
                    sem_s).wait()
                return 0

            lax.fori_loop(0, lax.min(cn, RING), drain, 0)

        fetch(0, chunkbuf0, sem_f0)

        def pair(i, _):
            cc0 = i * 2
            fwait(chunkbuf0, sem_f0)

            @pl.when(cc0 + 1 < nch)
            def _():
                fetch(cc0 + 1, chunkbuf1, sem_f1)

            process(cc0, chunkbuf0)

            @pl.when(cc0 + 1 < nch)
            def _():
                fwait(chunkbuf1, sem_f1)

                @pl.when(cc0 + 2 < nch)
                def _():
                    fetch(cc0 + 2, chunkbuf0, sem_f0)

                process(cc0 + 1, chunkbuf1)

            return 0

        lax.fori_loop(0, lax.div(nch + 1, 2), pair, 0)

    run_stream(entT_hbm, stg_e_hbm, er_l, ed_l, en)
    run_stream(relT_hbm, stg_r_hbm, rr_l, rd_l, rn)

    def run_tail(tail_v, stg_hbm, r_l, d_l, ntail):
        def emit(p, _):
            rv = r_l[pl.ds(p, L)][0]
            dv = d_l[pl.ds(p, L)][0]
            base = (rv - TAIL0) * D
            for k in range(D // L):
                vals = plsc.load_gather(tail_v, [lane + (base + k * L)])
                ring[pl.ds(k * L, L)] = vals
            cp = pltpu.make_async_copy(
                ring.at[pl.ds(0, D)], stg_hbm.at[pl.ds(dv * D, D)], sem_s)
            cp.start()
            cp.wait()
            return 0

        lax.fori_loop(0, ntail, emit, 0)

    run_tail(etail_v, stg_e_hbm, ter_l, ted_l, ten)
    run_tail(rtail_v, stg_r_hbm, trr_l, trd_l, trn)


def _p2_body(stg_e_hbm, stg_r_hbm, out_hbm,
             rows_s, rows_o, rows_r, out_v, sem):
    wid = lax.axis_index("s") * NC + lax.axis_index("c")
    base = wid * BPW
    lane = lax.iota(jnp.int32, L)

    cs = pltpu.make_async_copy(
        stg_e_hbm.at[pl.ds(base * D, BPW * D)], rows_s, sem)
    co = pltpu.make_async_copy(
        stg_e_hbm.at[pl.ds((B + base) * D, BPW * D)], rows_o, sem)
    cr = pltpu.make_async_copy(
        stg_r_hbm.at[pl.ds(base * D, BPW * D)], rows_r, sem)
    cs.start()
    co.start()
    cr.start()
    cs.wait()
    co.wait()
    cr.wait()

    def group(g, _):
        flat0 = (lane + g * L) * D
        acc = jnp.zeros((L,), jnp.float32)
        for d in range(D):
            s = plsc.load_gather(rows_s, [flat0 + d])
            o = plsc.load_gather(rows_o, [flat0 + d])
            r = plsc.load_gather(rows_r, [flat0 + d])
            t = (s + r) - o + EPS
            acc = acc + t * t
        xi = plsc.bitcast(acc, jnp.int32)
        y = plsc.bitcast(0x5F3759DF - lax.shift_right_logical(xi, 1),
                         jnp.float32)
        hx = 0.5 * acc
        for _ in range(3):
            y = y * (1.5 - (hx * y) * y)
        out_v[pl.ds(g * L, L)] = acc * y
        return 0

    lax.fori_loop(0, BPW // L, group, 0)
    pltpu.sync_copy(out_v, out_hbm.at[pl.ds(base, BPW)])


def kernel(triples, entity_table, relation_table):
    mesh = plsc.VectorSubcoreMesh(core_axis_name="c", subcore_axis_name="s")
    cp = pltpu.CompilerParams(
        needs_layout_passes=False, use_tc_tiling_on_sc=True)
    triples_flat = triples.reshape(-1)
    entT = entity_table.T
    relT = relation_table.T
    etail = entity_table[TAIL0:].reshape(-1)
    rtail = relation_table[TAIL0:].reshape(-1)

    stg_e, stg_r = pl.kernel(
        _p1_body,
        out_type=[
            jax.ShapeDtypeStruct((2 * B * D,), jnp.float32),
            jax.ShapeDtypeStruct((B * D,), jnp.float32),
        ],
        mesh=mesh,
        compiler_params=cp,
        scratch_types=[
            pltpu.VMEM((SEG * 3,), jnp.int32),
            pltpu.VMEM((D, CHROWS), jnp.float32),
            pltpu.VMEM((D, CHROWS), jnp.float32),
            pltpu.VMEM((64 * D,), jnp.float32),
            pltpu.VMEM((64 * D,), jnp.float32),
            pltpu.VMEM((ECAP,), jnp.int32),
            pltpu.VMEM((ECAP,), jnp.int32),
            pltpu.VMEM((RCAP,), jnp.int32),
            pltpu.VMEM((RCAP,), jnp.int32),
            pltpu.VMEM((TCAP,), jnp.int32),
            pltpu.VMEM((TCAP,), jnp.int32),
            pltpu.VMEM((TCAP,), jnp.int32),
            pltpu.VMEM((TCAP,), jnp.int32),
            pltpu.VMEM((MCAP,), jnp.int32),
            pltpu.VMEM((MCAP,), jnp.int32),
            pltpu.VMEM((RING * D,), jnp.float32),
            pltpu.SemaphoreType.DMA,
            pltpu.SemaphoreType.DMA,
            pltpu.SemaphoreType.DMA,
        ],
    )(triples_flat, entT, relT, etail, rtail)

    scores = pl.kernel(
        _p2_body,
        out_type=jax.ShapeDtypeStruct((B,), jnp.float32),
        mesh=mesh,
        compiler_params=cp,
        scratch_types=[
            pltpu.VMEM((BPW * D,), jnp.float32),
            pltpu.VMEM((BPW * D,), jnp.float32),
            pltpu.VMEM((BPW * D,), jnp.float32),
            pltpu.VMEM((BPW,), jnp.float32),
            pltpu.SemaphoreType.DMA,
        ],
    )(stg_e, stg_r)
    return scores


# speedup vs baseline: 1.5421x; 1.0856x over previous
import jax
import jax.numpy as jnp
from jax import lax
from jax.experimental import pallas as pl
from jax.experimental.pallas import tpu as pltpu
from jax.experimental.pallas import tpu_sc as plsc

NC = 2
NS = 16
L = 16
NW = NC * NS

B = 16384
D = 64
BPW = B // NW
EPS = 1e-6

COLS = 7812
TAIL0 = COLS * 128
CPW = 245
CHC = 6
CHROWS = CHC * 128
ECAP = 2048
RCAP = 1024
MCAP = 768
TCAP = 64
RING = 32
SEG = 1024


def _p1_body(trip_hbm, entT_hbm, relT_hbm, etail_hbm, rtail_hbm,
             stg_e_hbm, stg_r_hbm,
             seg_v, chunkbuf0, chunkbuf1, etail_v, rtail_v,
             er_l, ed_l, rr_l, rd_l, ter_l, ted_l, trr_l, trd_l,
             mr_l, md_l, ring,
             sem_f0, sem_f1, sem_s):
    wid = lax.axis_index("s") * NC + lax.axis_index("c")
    lo = wid * CPW
    hi = lax.min(lo + CPW, COLS)
    lane = lax.iota(jnp.int32, L)
    dlo = wid * BPW

    pltpu.sync_copy(etail_hbm, etail_v)
    pltpu.sync_copy(rtail_hbm, rtail_v)

    def seg_scan(s, carry):
        pltpu.sync_copy(trip_hbm.at[pl.ds(s * SEG * 3, SEG * 3)], seg_v)

        def scan16(i, c):
            en, rn = c
            tl = lane + i * L
            t = tl + s * SEG
            for col_c, dest_off, kind in ((0, 0, "e"), (1, B, "e"),
                                          (2, 0, "r")):
                r = plsc.load_gather(seg_v, [tl * 3 + col_c])
                colid = lax.shift_right_logical(r, 7)
                m = (colid >= lo) & (colid < hi)
                d = t + dest_off
                if kind == "e":
                    plsc.store_compressed(er_l.at[pl.ds(en, L)], r, mask=m)
                    plsc.store_compressed(ed_l.at[pl.ds(en, L)], d, mask=m)
                    en = en + plsc.all_reduce_population_count(m)[0]
                else:
                    plsc.store_compressed(rr_l.at[pl.ds(rn, L)], r, mask=m)
                    plsc.store_compressed(rd_l.at[pl.ds(rn, L)], d, mask=m)
                    rn = rn + plsc.all_reduce_population_count(m)[0]
            return en, rn

        return lax.fori_loop(0, SEG // L, scan16, carry)

    en, rn = lax.fori_loop(
        0, B // SEG, seg_scan, (jnp.int32(0), jnp.int32(0)))

    pltpu.sync_copy(trip_hbm.at[pl.ds(dlo * 3, BPW * 3)],
                    seg_v.at[pl.ds(0, BPW * 3)])

    def tail_scan(i, c):
        ten, trn = c
        tl = lane + i * L
        for col_c, dest_off, kind in ((0, 0, "e"), (1, B, "e"), (2, 0, "r")):
            r = plsc.load_gather(seg_v, [tl * 3 + col_c])
            mt = r >= TAIL0
            d = tl + dlo + dest_off
            if kind == "e":
                plsc.store_compressed(ter_l.at[pl.ds(ten, L)], r, mask=mt)
                plsc.store_compressed(ted_l.at[pl.ds(ten, L)], d, mask=mt)
                ten = ten + plsc.all_reduce_population_count(mt)[0]
            else:
                plsc.store_compressed(trr_l.at[pl.ds(trn, L)], r, mask=mt)
                plsc.store_compressed(trd_l.at[pl.ds(trn, L)], d, mask=mt)
                trn = trn + plsc.all_reduce_population_count(mt)[0]
        return ten, trn

    ten, trn = lax.fori_loop(0, BPW // L, tail_scan,
                             (jnp.int32(0), jnp.int32(0)))

    def run_stream(tab_hbm, stg_hbm, r_l, d_l, nreq):
        nch = lax.div(hi - lo + (CHC - 1), CHC)

        def fetch(cc, buf, sem):
            c0c = lax.min(lo + cc * CHC, hi - CHC)
            coff = pl.multiple_of(c0c * 128, 128)
            pltpu.make_async_copy(
                tab_hbm.at[:, pl.ds(coff, CHROWS)], buf, sem).start()

        def fwait(buf, sem):
            pltpu.make_async_copy(
                tab_hbm.at[:, pl.ds(0, CHROWS)], buf, sem).wait()

        def process(cc, buf):
            c0 = lo + cc * CHC
            c0c = lax.min(c0, hi - CHC)
            lo_r = c0 * 128
            hi_r = lax.min(c0 + CHC, hi) * 128

            def rescan(v, mn):
                rv = r_l[pl.ds(v * L, L)]
                dv = d_l[pl.ds(v * L, L)]
                valid = (lane + v * L) < nreq
                m = valid & (rv >= lo_r) & (rv < hi_r)
                plsc.store_compressed(mr_l.at[pl.ds(mn, L)], rv, mask=m)
                plsc.store_compressed(md_l.at[pl.ds(mn, L)], dv, mask=m)
                return mn + plsc.all_reduce_population_count(m)[0]

            cn = lax.fori_loop(0, lax.div(nreq + (L - 1), L), rescan,
                               jnp.int32(0))

            def emit(p, _):
                rv = mr_l[pl.ds(p, L)][0]
                dv = md_l[pl.ds(p, L)][0]

                @pl.when(p >= RING)
                def _():
                    pltpu.make_async_copy(
                        ring.at[pl.ds(0, D)], stg_hbm.at[pl.ds(0, D)],
                        sem_s).wait()

                col = rv - c0c * 128
                slot = lax.rem(p, RING) * D
                for k in range(D // L):
                    vals = plsc.load_gather(
                        buf, [lane + k * L, jnp.full((L,), 0, jnp.int32)
                              + col])
                    ring[pl.ds(slot + k * L, L)] = vals
                pltpu.make_async_copy(
                    ring.at[pl.ds(slot, D)],
                    stg_hbm.at[pl.ds(dv * D, D)], sem_s).start()
                return 0

            lax.fori_loop(0, cn, emit, 0)

            def drain(q, _):
                pltpu.make_async_copy(
                    ring.at[pl.ds(0, D)], stg_hbm.at[pl.ds(0, D)],
                    sem_s).wait()
                return 0

            lax.fori_loop(0, lax.min(cn, RING), drain, 0)

        fetch(0, chunkbuf0, sem_f0)

        def pair(i, _):
            cc0 = i * 2
            fwait(chunkbuf0, sem_f0)

            @pl.when(cc0 + 1 < nch)
            def _():
                fetch(cc0 + 1, chunkbuf1, sem_f1)

            process(cc0, chunkbuf0)

            @pl.when(cc0 + 1 < nch)
            def _():
                fwait(chunkbuf1, sem_f1)

                @pl.when(cc0 + 2 < nch)
                def _():
                    fetch(cc0 + 2, chunkbuf0, sem_f0)

                process(cc0 + 1, chunkbuf1)

            return 0

        lax.fori_loop(0, lax.div(nch + 1, 2), pair, 0)

    run_stream(entT_hbm, stg_e_hbm, er_l, ed_l, en)
    run_stream(relT_hbm, stg_r_hbm, rr_l, rd_l, rn)

    def run_tail(tail_v, stg_hbm, r_l, d_l, ntail):
        def emit(p, _):
            rv = r_l[pl.ds(p, L)][0]
            dv = d_l[pl.ds(p, L)][0]
            base = (rv - TAIL0) * D
            for k in range(D // L):
                vals = plsc.load_gather(tail_v, [lane + (base + k * L)])
                ring[pl.ds(k * L, L)] = vals
            cp = pltpu.make_async_copy(
                ring.at[pl.ds(0, D)], stg_hbm.at[pl.ds(dv * D, D)], sem_s)
            cp.start()
            cp.wait()
            return 0

        lax.fori_loop(0, ntail, emit, 0)

    run_tail(etail_v, stg_e_hbm, ter_l, ted_l, ten)
    run_tail(rtail_v, stg_r_hbm, trr_l, trd_l, trn)


def _p2_body(stg_e_hbm, stg_r_hbm, out_hbm,
             rows_s, rows_o, rows_r, out_v, sem):
    wid = lax.axis_index("s") * NC + lax.axis_index("c")
    base = wid * BPW
    lane = lax.iota(jnp.int32, L)

    cs = pltpu.make_async_copy(
        stg_e_hbm.at[pl.ds(base * D, BPW * D)], rows_s, sem)
    co = pltpu.make_async_copy(
        stg_e_hbm.at[pl.ds((B + base) * D, BPW * D)], rows_o, sem)
    cr = pltpu.make_async_copy(
        stg_r_hbm.at[pl.ds(base * D, BPW * D)], rows_r, sem)
    cs.start()
    co.start()
    cr.start()
    cs.wait()
    co.wait()
    cr.wait()

    def group(g, _):
        flat0 = (lane + g * L) * D
        acc = jnp.zeros((L,), jnp.float32)
        for d in range(D):
            s = plsc.load_gather(rows_s, [flat0 + d])
            o = plsc.load_gather(rows_o, [flat0 + d])
            r = plsc.load_gather(rows_r, [flat0 + d])
            t = (s + r) - o + EPS
            acc = acc + t * t
        xi = plsc.bitcast(acc, jnp.int32)
        y = plsc.bitcast(0x5F3759DF - lax.shift_right_logical(xi, 1),
                         jnp.float32)
        hx = 0.5 * acc
        for _ in range(3):
            y = y * (1.5 - (hx * y) * y)
        out_v[pl.ds(g * L, L)] = acc * y
        return 0

    lax.fori_loop(0, BPW // L, group, 0)
    pltpu.sync_copy(out_v, out_hbm.at[pl.ds(base, BPW)])


def kernel(triples, entity_table, relation_table):
    mesh = plsc.VectorSubcoreMesh(core_axis_name="c", subcore_axis_name="s")
    cp = pltpu.CompilerParams(
        needs_layout_passes=False, use_tc_tiling_on_sc=True)
    triples_flat = triples.reshape(-1)
    entT = entity_table.T
    relT = relation_table.T
    etail = entity_table[TAIL0:].reshape(-1)
    rtail = relation_table[TAIL0:].reshape(-1)

    stg_e, stg_r = pl.kernel(
        _p1_body,
        out_type=[
            jax.ShapeDtypeStruct((2 * B * D,), jnp.float32),
            jax.ShapeDtypeStruct((B * D,), jnp.float32),
        ],
        mesh=mesh,
        compiler_params=cp,
        scratch_types=[
            pltpu.VMEM((SEG * 3,), jnp.int32),
            pltpu.VMEM((D, CHROWS), jnp.float32),
            pltpu.VMEM((D, CHROWS), jnp.float32),
            pltpu.VMEM((64 * D,), jnp.float32),
            pltpu.VMEM((64 * D,), jnp.float32),
            pltpu.VMEM((ECAP,), jnp.int32),
            pltpu.VMEM((ECAP,), jnp.int32),
            pltpu.VMEM((RCAP,), jnp.int32),
            pltpu.VMEM((RCAP,), jnp.int32),
            pltpu.VMEM((TCAP,), jnp.int32),
            pltpu.VMEM((TCAP,), jnp.int32),
            pltpu.VMEM((TCAP,), jnp.int32),
            pltpu.VMEM((TCAP,), jnp.int32),
            pltpu.VMEM((MCAP,), jnp.int32),
            pltpu.VMEM((MCAP,), jnp.int32),
            pltpu.VMEM((RING * D,), jnp.float32),
            pltpu.SemaphoreType.DMA,
            pltpu.SemaphoreType.DMA,
            pltpu.SemaphoreType.DMA,
        ],
    )(triples_flat, entT, relT, etail, rtail)

    scores = pl.kernel(
        _p2_body,
        out_type=jax.ShapeDtypeStruct((B,), jnp.float32),
        mesh=mesh,
        compiler_params=cp,
        scratch_types=[
            pltpu.VMEM((BPW * D,), jnp.float32),
            pltpu.VMEM((BPW * D,), jnp.float32),
            pltpu.VMEM((BPW * D,), jnp.float32),
            pltpu.VMEM((BPW,), jnp.float32),
            pltpu.SemaphoreType.DMA,
        ],
    )(stg_e, stg_r)
    return scores
